# baseline (speedup 1.0000x reference)
import functools

import jax
import jax.numpy as jnp
from jax import lax
from jax.experimental import pallas as pl
from jax.experimental.pallas import tpu as pltpu
from jax.experimental.pallas import tpu_sc as plsc

F = 100
SEG = 64
B = 4096
D = F * SEG
RED = 50
L = 16
FP = 112
RP = 64
NC = 2
NS = 16
NW = NC * NS
ROWS_PER_W = B // NW
G = 16
NGROUPS = ROWS_PER_W // G

_IN_BOUNDS = lax.GatherScatterMode.PROMISE_IN_BOUNDS

_GATHER_DNUMS = lax.GatherDimensionNumbers(
    offset_dims=(), collapsed_slice_dims=(0,), start_index_map=(0,))


def _lane_perm(v, idx_vec):
    return lax.gather(v, idx_vec.reshape(L, 1), _GATHER_DNUMS, (1,),
                      mode=_IN_BOUNDS)


def _bcast_lane(v, t, lane):
    return _lane_perm(v, lane * 0 + t)


def _lane_sum(v, lane):
    for k in (8, 4, 2, 1):
        v = v + _lane_perm(v, (lane + k) & (L - 1))
    return v


def _sc_body(x_hbm, w1_hbm, w2_hbm, out_hbm, xbuf, w1_v, w2_v, xxT, hT, sT):
    wid = lax.axis_index("s") * NC + lax.axis_index("c")
    row_base = wid * ROWS_PER_W

    pltpu.sync_copy(w1_hbm, w1_v)
    pltpu.sync_copy(w2_hbm, w2_v)

    lane = lax.iota(jnp.int32, L)
    zero_v = (lane * 0).astype(jnp.float32)
    lo8 = lane < 8
    rot8 = (lane + 8) & (L - 1)
    rot4 = (lane + 4) & (L - 1)
    rot2 = (lane + 2) & (L - 1)
    rot1 = (lane + 1) & (L - 1)
    pair_mask = (lane == 0) | (lane == 8)
    pair_off = jnp.where(lo8, 0, L)

    for i in range(F, FP):
        xxT[pl.ds(i * L, L)] = zero_v
    for j in range(RED, RP):
        hT[pl.ds(j * L, L)] = zero_v

    def group_body(g, carry):
        row0 = row_base + g * G
        pltpu.sync_copy(x_hbm.at[pl.ds(row0, G)], xbuf)

        for r in range(G):
            def pair_body(f2, c, r=r):
                base = f2 * 2 * SEG
                pa = ((xbuf[r, pl.ds(base, L)]
                       + xbuf[r, pl.ds(base + L, L)])
                      + (xbuf[r, pl.ds(base + 2 * L, L)]
                         + xbuf[r, pl.ds(base + 3 * L, L)]))
                pb = ((xbuf[r, pl.ds(base + 4 * L, L)]
                       + xbuf[r, pl.ds(base + 5 * L, L)])
                      + (xbuf[r, pl.ds(base + 6 * L, L)]
                         + xbuf[r, pl.ds(base + 7 * L, L)]))
                w = (jnp.where(lo8, pa, _lane_perm(pb, rot8))
                     + jnp.where(lo8, _lane_perm(pa, rot8), pb))
                w = w + _lane_perm(w, rot4)
                w = w + _lane_perm(w, rot2)
                w = w + _lane_perm(w, rot1)
                idx = (lane * 0 + (f2 * 2 * L + r)) + pair_off
                plsc.store_scatter(xxT, [idx], w, mask=pair_mask)
                return c
            lax.fori_loop(0, F // 2, pair_body, 0, unroll=4)

        def mlp1_body(j, c):
            accs = [zero_v] * 4
            for fc in range(FP // L):
                wv = w1_v[pl.ds(j * FP + fc * L, L)]
                for t in range(L):
                    accs[t & 3] = accs[t & 3] + _bcast_lane(wv, t, lane) * xxT[
                        pl.ds((fc * L + t) * L, L)]
            acc = (accs[0] + accs[1]) + (accs[2] + accs[3])
            hT[pl.ds(j * L, L)] = jnp.maximum(acc, 0.0)
            return c
        lax.fori_loop(0, RED, mlp1_body, 0)

        def mlp2_body(i, c):
            accs = [zero_v] * 4
            for jc in range(RP // L):
                wv = w2_v[pl.ds(i * RP + jc * L, L)]
                for t in range(L):
                    accs[t & 3] = accs[t & 3] + _bcast_lane(wv, t, lane) * hT[
                        pl.ds((jc * L + t) * L, L)]
            acc = (accs[0] + accs[1]) + (accs[2] + accs[3])
            sT[pl.ds(i * L, L)] = 1.0 / (1.0 + jnp.exp(-acc))
            return c
        lax.fori_loop(0, F, mlp2_body, 0)

        for r in range(G):
            def scale_body(f, c, r=r):
                sc = _bcast_lane(sT[pl.ds(f * L, L)], r, lane)
                base = f * SEG
                for t in range(4):
                    o = base + t * L
                    xbuf[r, pl.ds(o, L)] = xbuf[r, pl.ds(o, L)] * sc
                return c
            lax.fori_loop(0, F, scale_body, 0, unroll=2)

        pltpu.sync_copy(xbuf, out_hbm.at[pl.ds(row0, G)])
        return carry

    lax.fori_loop(0, NGROUPS, group_body, 0)


@functools.cache
def _sc_call():
    return pl.kernel(
        _sc_body,
        out_type=jax.ShapeDtypeStruct((B, D), jnp.float32),
        mesh=plsc.VectorSubcoreMesh(core_axis_name="c", subcore_axis_name="s",
                                    num_cores=NC, num_subcores=NS),
        compiler_params=pltpu.CompilerParams(needs_layout_passes=False),
        scratch_types=[
            pltpu.VMEM((G, D), jnp.float32),
            pltpu.VMEM((RED * FP,), jnp.float32),
            pltpu.VMEM((F * RP,), jnp.float32),
            pltpu.VMEM((FP * L,), jnp.float32),
            pltpu.VMEM((RP * L,), jnp.float32),
            pltpu.VMEM((F * L,), jnp.float32),
        ],
    )


def kernel(x, W1, W2):
    w1p = jnp.zeros((RED, FP), jnp.float32).at[:, :F].set(W1 * (1.0 / SEG))
    w2p = jnp.zeros((F, RP), jnp.float32).at[:, :RED].set(W2)
    return _sc_call()(x, w1p.reshape(-1), w2p.reshape(-1))


# speedup vs baseline: 1.9561x; 1.9561x over previous
import functools

import jax
import jax.numpy as jnp
from jax import lax
from jax.experimental import pallas as pl
from jax.experimental.pallas import tpu as pltpu
from jax.experimental.pallas import tpu_sc as plsc

F = 100
SEG = 64
B = 4096
D = F * SEG
RED = 50
L = 16
FP = 112
RP = 64
NC = 2
NS = 16
NW = NC * NS
G = 16
B_SC = 512
B_TC = B - B_SC
ROWS_PER_W = B_SC // NW
NGROUPS = ROWS_PER_W // G
TILE_B = 256

_IN_BOUNDS = lax.GatherScatterMode.PROMISE_IN_BOUNDS

_GATHER_DNUMS = lax.GatherDimensionNumbers(
    offset_dims=(), collapsed_slice_dims=(0,), start_index_map=(0,))


def _lane_perm(v, idx_vec):
    return lax.gather(v, idx_vec.reshape(L, 1), _GATHER_DNUMS, (1,),
                      mode=_IN_BOUNDS)


def _bcast_lane(v, t, lane):
    return _lane_perm(v, lane * 0 + t)


def _lane_sum(v, lane):
    for k in (8, 4, 2, 1):
        v = v + _lane_perm(v, (lane + k) & (L - 1))
    return v


def _sc_body(x_hbm, w1_hbm, w2_hbm, out_hbm, xbuf, w1_v, w2_v, xxT, hT, sT):
    wid = lax.axis_index("s") * NC + lax.axis_index("c")
    row_base = wid * ROWS_PER_W

    pltpu.sync_copy(w1_hbm, w1_v)
    pltpu.sync_copy(w2_hbm, w2_v)

    lane = lax.iota(jnp.int32, L)
    zero_v = (lane * 0).astype(jnp.float32)
    lo8 = lane < 8
    rot8 = (lane + 8) & (L - 1)
    rot4 = (lane + 4) & (L - 1)
    rot2 = (lane + 2) & (L - 1)
    rot1 = (lane + 1) & (L - 1)
    pair_mask = (lane == 0) | (lane == 8)
    pair_off = jnp.where(lo8, 0, L)

    for i in range(F, FP):
        xxT[pl.ds(i * L, L)] = zero_v
    for j in range(RED, RP):
        hT[pl.ds(j * L, L)] = zero_v

    def group_body(g, carry):
        row0 = row_base + g * G
        pltpu.sync_copy(x_hbm.at[pl.ds(row0, G)], xbuf)

        for r in range(G):
            def pair_body(f2, c, r=r):
                base = f2 * 2 * SEG
                pa = ((xbuf[r, pl.ds(base, L)]
                       + xbuf[r, pl.ds(base + L, L)])
                      + (xbuf[r, pl.ds(base + 2 * L, L)]
                         + xbuf[r, pl.ds(base + 3 * L, L)]))
                pb = ((xbuf[r, pl.ds(base + 4 * L, L)]
                       + xbuf[r, pl.ds(base + 5 * L, L)])
                      + (xbuf[r, pl.ds(base + 6 * L, L)]
                         + xbuf[r, pl.ds(base + 7 * L, L)]))
                w = (jnp.where(lo8, pa, _lane_perm(pb, rot8))
                     + jnp.where(lo8, _lane_perm(pa, rot8), pb))
                w = w + _lane_perm(w, rot4)
                w = w + _lane_perm(w, rot2)
                w = w + _lane_perm(w, rot1)
                idx = (lane * 0 + (f2 * 2 * L + r)) + pair_off
                plsc.store_scatter(xxT, [idx], w, mask=pair_mask)
                return c
            lax.fori_loop(0, F // 2, pair_body, 0, unroll=4)

        def mlp1_body(j, c):
            accs = [zero_v] * 4
            for fc in range(FP // L):
                wv = w1_v[pl.ds(j * FP + fc * L, L)]
                for t in range(L):
                    accs[t & 3] = accs[t & 3] + _bcast_lane(wv, t, lane) * xxT[
                        pl.ds((fc * L + t) * L, L)]
            acc = (accs[0] + accs[1]) + (accs[2] + accs[3])
            hT[pl.ds(j * L, L)] = jnp.maximum(acc, 0.0)
            return c
        lax.fori_loop(0, RED, mlp1_body, 0)

        def mlp2_body(i, c):
            accs = [zero_v] * 4
            for jc in range(RP // L):
                wv = w2_v[pl.ds(i * RP + jc * L, L)]
                for t in range(L):
                    accs[t & 3] = accs[t & 3] + _bcast_lane(wv, t, lane) * hT[
                        pl.ds((jc * L + t) * L, L)]
            acc = (accs[0] + accs[1]) + (accs[2] + accs[3])
            sT[pl.ds(i * L, L)] = 1.0 / (1.0 + jnp.exp(-acc))
            return c
        lax.fori_loop(0, F, mlp2_body, 0)

        for r in range(G):
            def scale_body(f, c, r=r):
                sc = _bcast_lane(sT[pl.ds(f * L, L)], r, lane)
                base = f * SEG
                for t in range(4):
                    o = base + t * L
                    xbuf[r, pl.ds(o, L)] = xbuf[r, pl.ds(o, L)] * sc
                return c
            lax.fori_loop(0, F, scale_body, 0, unroll=2)

        pltpu.sync_copy(xbuf, out_hbm.at[pl.ds(row0, G)])
        return carry

    lax.fori_loop(0, NGROUPS, group_body, 0)


@functools.cache
def _sc_call():
    return pl.kernel(
        _sc_body,
        out_type=jax.ShapeDtypeStruct((B_SC, D), jnp.float32),
        mesh=plsc.VectorSubcoreMesh(core_axis_name="c", subcore_axis_name="s",
                                    num_cores=NC, num_subcores=NS),
        compiler_params=pltpu.CompilerParams(needs_layout_passes=False),
        scratch_types=[
            pltpu.VMEM((G, D), jnp.float32),
            pltpu.VMEM((RED * FP,), jnp.float32),
            pltpu.VMEM((F * RP,), jnp.float32),
            pltpu.VMEM((FP * L,), jnp.float32),
            pltpu.VMEM((RP * L,), jnp.float32),
            pltpu.VMEM((F * L,), jnp.float32),
        ],
    )


import numpy as np

_SM = np.repeat(np.eye(F, dtype=np.float32), SEG, axis=0) * (1.0 / SEG)
_RM = np.repeat(np.eye(F, dtype=np.float32), SEG, axis=1)


def _tc_body(x_ref, w1t_ref, w2t_ref, sm_ref, rm_ref, o_ref):
    xb = x_ref[...]
    x_hi = xb.astype(jnp.bfloat16)
    x_lo = (xb - x_hi.astype(jnp.float32)).astype(jnp.bfloat16)
    sm = sm_ref[...]
    xx = (jnp.dot(x_hi, sm, preferred_element_type=jnp.float32)
          + jnp.dot(x_lo, sm, preferred_element_type=jnp.float32))
    h = jnp.maximum(jnp.dot(xx, w1t_ref[...],
                            preferred_element_type=jnp.float32), 0.0)
    s = jax.nn.sigmoid(jnp.dot(h, w2t_ref[...],
                               preferred_element_type=jnp.float32))
    s_hi = s.astype(jnp.bfloat16)
    s_lo = (s - s_hi.astype(jnp.float32)).astype(jnp.bfloat16)
    rm = rm_ref[...]
    s_rep = (jnp.dot(s_hi, rm, preferred_element_type=jnp.float32)
             + jnp.dot(s_lo, rm, preferred_element_type=jnp.float32))
    o_ref[...] = xb * s_rep


def _tc_call(x_tc, W1, W2):
    w1t = W1.T
    w2t = W2.T
    sm = jnp.asarray(_SM, dtype=jnp.bfloat16)
    rm = jnp.asarray(_RM, dtype=jnp.bfloat16)
    return pl.pallas_call(
        _tc_body,
        grid=(B_TC // TILE_B,),
        in_specs=[
            pl.BlockSpec((TILE_B, D), lambda i: (i, 0)),
            pl.BlockSpec((F, RED), lambda i: (0, 0)),
            pl.BlockSpec((RED, F), lambda i: (0, 0)),
            pl.BlockSpec((D, F), lambda i: (0, 0)),
            pl.BlockSpec((F, D), lambda i: (0, 0)),
        ],
        out_specs=pl.BlockSpec((TILE_B, D), lambda i: (i, 0)),
        out_shape=jax.ShapeDtypeStruct((B_TC, D), jnp.float32),
    )(x_tc, w1t, w2t, sm, rm)


def kernel(x, W1, W2):
    w1p = jnp.zeros((RED, FP), jnp.float32).at[:, :F].set(W1 * (1.0 / SEG))
    w2p = jnp.zeros((F, RP), jnp.float32).at[:, :RED].set(W2)
    out_sc = _sc_call()(x[:B_SC], w1p.reshape(-1), w2p.reshape(-1))
    out_tc = _tc_call(x[B_SC:], W1, W2)
    return jnp.concatenate([out_sc, out_tc], axis=0)


# speedup vs baseline: 3.8488x; 1.9676x over previous
import functools

import jax
import jax.numpy as jnp
from jax import lax
from jax.experimental import pallas as pl
from jax.experimental.pallas import tpu as pltpu
from jax.experimental.pallas import tpu_sc as plsc

F = 100
SEG = 64
B = 4096
D = F * SEG
RED = 50
L = 16
FP = 112
RP = 64
NC = 2
NS = 16
NW = NC * NS
G = 16
B_SC = 512
B_TC = B - B_SC
ROWS_PER_W = B_SC // NW
NGROUPS = ROWS_PER_W // G
TILE_B = 256

_IN_BOUNDS = lax.GatherScatterMode.PROMISE_IN_BOUNDS

_GATHER_DNUMS = lax.GatherDimensionNumbers(
    offset_dims=(), collapsed_slice_dims=(0,), start_index_map=(0,))


def _lane_perm(v, idx_vec):
    return lax.gather(v, idx_vec.reshape(L, 1), _GATHER_DNUMS, (1,),
                      mode=_IN_BOUNDS)


def _bcast_lane(v, t, lane):
    return _lane_perm(v, lane * 0 + t)


def _lane_sum(v, lane):
    for k in (8, 4, 2, 1):
        v = v + _lane_perm(v, (lane + k) & (L - 1))
    return v


def _sc_body(x_hbm, w1_hbm, w2_hbm, out_hbm, xbuf, w1_v, w2_v, xxT, hT, sT):
    wid = lax.axis_index("s") * NC + lax.axis_index("c")
    row_base = wid * ROWS_PER_W

    pltpu.sync_copy(w1_hbm, w1_v)
    pltpu.sync_copy(w2_hbm, w2_v)

    lane = lax.iota(jnp.int32, L)
    zero_v = (lane * 0).astype(jnp.float32)
    lo8 = lane < 8
    rot8 = (lane + 8) & (L - 1)
    rot4 = (lane + 4) & (L - 1)
    rot2 = (lane + 2) & (L - 1)
    rot1 = (lane + 1) & (L - 1)
    pair_mask = (lane == 0) | (lane == 8)
    pair_off = jnp.where(lo8, 0, L)

    for i in range(F, FP):
        xxT[pl.ds(i * L, L)] = zero_v
    for j in range(RED, RP):
        hT[pl.ds(j * L, L)] = zero_v

    def group_body(g, carry):
        row0 = row_base + g * G
        pltpu.sync_copy(x_hbm.at[pl.ds(row0, G)], xbuf)

        for r in range(G):
            def pair_body(f2, c, r=r):
                base = f2 * 2 * SEG
                pa = ((xbuf[r, pl.ds(base, L)]
                       + xbuf[r, pl.ds(base + L, L)])
                      + (xbuf[r, pl.ds(base + 2 * L, L)]
                         + xbuf[r, pl.ds(base + 3 * L, L)]))
                pb = ((xbuf[r, pl.ds(base + 4 * L, L)]
                       + xbuf[r, pl.ds(base + 5 * L, L)])
                      + (xbuf[r, pl.ds(base + 6 * L, L)]
                         + xbuf[r, pl.ds(base + 7 * L, L)]))
                w = (jnp.where(lo8, pa, _lane_perm(pb, rot8))
                     + jnp.where(lo8, _lane_perm(pa, rot8), pb))
                w = w + _lane_perm(w, rot4)
                w = w + _lane_perm(w, rot2)
                w = w + _lane_perm(w, rot1)
                idx = (lane * 0 + (f2 * 2 * L + r)) + pair_off
                plsc.store_scatter(xxT, [idx], w, mask=pair_mask)
                return c
            lax.fori_loop(0, F // 2, pair_body, 0, unroll=4)

        def mlp1_body(j, c):
            accs = [zero_v] * 4
            for fc in range(FP // L):
                wv = w1_v[pl.ds(j * FP + fc * L, L)]
                for t in range(L):
                    accs[t & 3] = accs[t & 3] + _bcast_lane(wv, t, lane) * xxT[
                        pl.ds((fc * L + t) * L, L)]
            acc = (accs[0] + accs[1]) + (accs[2] + accs[3])
            hT[pl.ds(j * L, L)] = jnp.maximum(acc, 0.0)
            return c
        lax.fori_loop(0, RED, mlp1_body, 0)

        def mlp2_body(i, c):
            accs = [zero_v] * 4
            for jc in range(RP // L):
                wv = w2_v[pl.ds(i * RP + jc * L, L)]
                for t in range(L):
                    accs[t & 3] = accs[t & 3] + _bcast_lane(wv, t, lane) * hT[
                        pl.ds((jc * L + t) * L, L)]
            acc = (accs[0] + accs[1]) + (accs[2] + accs[3])
            sT[pl.ds(i * L, L)] = 1.0 / (1.0 + jnp.exp(-acc))
            return c
        lax.fori_loop(0, F, mlp2_body, 0)

        for r in range(G):
            def scale_body(f, c, r=r):
                sc = _bcast_lane(sT[pl.ds(f * L, L)], r, lane)
                base = f * SEG
                for t in range(4):
                    o = base + t * L
                    xbuf[r, pl.ds(o, L)] = xbuf[r, pl.ds(o, L)] * sc
                return c
            lax.fori_loop(0, F, scale_body, 0, unroll=2)

        pltpu.sync_copy(xbuf, out_hbm.at[pl.ds(row0, G)])
        return carry

    lax.fori_loop(0, NGROUPS, group_body, 0)


@functools.cache
def _sc_call():
    return pl.kernel(
        _sc_body,
        out_type=jax.ShapeDtypeStruct((B_SC, D), jnp.float32),
        mesh=plsc.VectorSubcoreMesh(core_axis_name="c", subcore_axis_name="s",
                                    num_cores=NC, num_subcores=NS),
        compiler_params=pltpu.CompilerParams(needs_layout_passes=False),
        scratch_types=[
            pltpu.VMEM((G, D), jnp.float32),
            pltpu.VMEM((RED * FP,), jnp.float32),
            pltpu.VMEM((F * RP,), jnp.float32),
            pltpu.VMEM((FP * L,), jnp.float32),
            pltpu.VMEM((RP * L,), jnp.float32),
            pltpu.VMEM((F * L,), jnp.float32),
        ],
    )


import numpy as np

_SM = np.repeat(np.eye(F, dtype=np.float32), SEG, axis=0) * (1.0 / SEG)
_RM = np.repeat(np.eye(F, dtype=np.float32), SEG, axis=1)


def _tc_body(x_ref, w1t_ref, w2t_ref, sm_ref, rm_ref, o_ref):
    xb = x_ref[...]
    x_hi = xb.astype(jnp.bfloat16)
    x_lo = (xb - x_hi.astype(jnp.float32)).astype(jnp.bfloat16)
    sm = sm_ref[...]
    xx = (jnp.dot(x_hi, sm, preferred_element_type=jnp.float32)
          + jnp.dot(x_lo, sm, preferred_element_type=jnp.float32))
    h = jnp.maximum(jnp.dot(xx, w1t_ref[...],
                            preferred_element_type=jnp.float32), 0.0)
    s = jax.nn.sigmoid(jnp.dot(h, w2t_ref[...],
                               preferred_element_type=jnp.float32))
    s_hi = s.astype(jnp.bfloat16)
    s_lo = (s - s_hi.astype(jnp.float32)).astype(jnp.bfloat16)
    rm = rm_ref[...]
    s_rep = (jnp.dot(s_hi, rm, preferred_element_type=jnp.float32)
             + jnp.dot(s_lo, rm, preferred_element_type=jnp.float32))
    o_ref[...] = xb * s_rep


def _tc_call(x, W1, W2):
    w1t = W1.T
    w2t = W2.T
    sm = jnp.asarray(_SM, dtype=jnp.bfloat16)
    rm = jnp.asarray(_RM, dtype=jnp.bfloat16)
    off = B_SC // TILE_B
    return pl.pallas_call(
        _tc_body,
        grid=(B_TC // TILE_B,),
        in_specs=[
            pl.BlockSpec((TILE_B, D), lambda i: (i + off, 0)),
            pl.BlockSpec((F, RED), lambda i: (0, 0)),
            pl.BlockSpec((RED, F), lambda i: (0, 0)),
            pl.BlockSpec((D, F), lambda i: (0, 0)),
            pl.BlockSpec((F, D), lambda i: (0, 0)),
        ],
        out_specs=pl.BlockSpec((TILE_B, D), lambda i: (i + off, 0)),
        out_shape=jax.ShapeDtypeStruct((B, D), jnp.float32),
    )(x, w1t, w2t, sm, rm)


def kernel(x, W1, W2):
    w1p = jnp.zeros((RED, FP), jnp.float32).at[:, :F].set(W1 * (1.0 / SEG))
    w2p = jnp.zeros((F, RP), jnp.float32).at[:, :RED].set(W2)
    out_tc = _tc_call(x, W1, W2)
    out_sc = _sc_call()(x[:B_SC], w1p.reshape(-1), w2p.reshape(-1))
    return lax.dynamic_update_slice(out_tc, out_sc, (0, 0))


# speedup vs baseline: 4.1806x; 1.0862x over previous
import functools

import jax
import jax.numpy as jnp
from jax import lax
from jax.experimental import pallas as pl
from jax.experimental.pallas import tpu as pltpu
from jax.experimental.pallas import tpu_sc as plsc

F = 100
SEG = 64
B = 4096
D = F * SEG
RED = 50
L = 16
FP = 112
RP = 64
NC = 2
NS = 16
NW = NC * NS
G = 16
B_SC = 512
B_TC = B - B_SC
ROWS_PER_W = B_SC // NW
NGROUPS = ROWS_PER_W // G
TILE_B = 256

_IN_BOUNDS = lax.GatherScatterMode.PROMISE_IN_BOUNDS

_GATHER_DNUMS = lax.GatherDimensionNumbers(
    offset_dims=(), collapsed_slice_dims=(0,), start_index_map=(0,))


def _lane_perm(v, idx_vec):
    return lax.gather(v, idx_vec.reshape(L, 1), _GATHER_DNUMS, (1,),
                      mode=_IN_BOUNDS)


def _bcast_lane(v, t, lane):
    return _lane_perm(v, lane * 0 + t)


def _lane_sum(v, lane):
    for k in (8, 4, 2, 1):
        v = v + _lane_perm(v, (lane + k) & (L - 1))
    return v


def _sc_body(x_hbm, w1_hbm, w2_hbm, out_hbm, xbuf, w1_v, w2_v, xxT, hT, sT):
    wid = lax.axis_index("s") * NC + lax.axis_index("c")
    row_base = wid * ROWS_PER_W

    pltpu.sync_copy(w1_hbm, w1_v)
    pltpu.sync_copy(w2_hbm, w2_v)

    lane = lax.iota(jnp.int32, L)
    zero_v = (lane * 0).astype(jnp.float32)
    lo8 = lane < 8
    rot8 = (lane + 8) & (L - 1)
    rot4 = (lane + 4) & (L - 1)
    rot2 = (lane + 2) & (L - 1)
    rot1 = (lane + 1) & (L - 1)
    pair_mask = (lane == 0) | (lane == 8)
    pair_off = jnp.where(lo8, 0, L)

    for i in range(F, FP):
        xxT[pl.ds(i * L, L)] = zero_v
    for j in range(RED, RP):
        hT[pl.ds(j * L, L)] = zero_v

    def group_body(g, carry):
        row0 = row_base + g * G
        pltpu.sync_copy(x_hbm.at[pl.ds(row0, G)], xbuf)

        for r in range(G):
            def pair_body(f2, c, r=r):
                base = f2 * 2 * SEG
                pa = ((xbuf[r, pl.ds(base, L)]
                       + xbuf[r, pl.ds(base + L, L)])
                      + (xbuf[r, pl.ds(base + 2 * L, L)]
                         + xbuf[r, pl.ds(base + 3 * L, L)]))
                pb = ((xbuf[r, pl.ds(base + 4 * L, L)]
                       + xbuf[r, pl.ds(base + 5 * L, L)])
                      + (xbuf[r, pl.ds(base + 6 * L, L)]
                         + xbuf[r, pl.ds(base + 7 * L, L)]))
                w = (jnp.where(lo8, pa, _lane_perm(pb, rot8))
                     + jnp.where(lo8, _lane_perm(pa, rot8), pb))
                w = w + _lane_perm(w, rot4)
                w = w + _lane_perm(w, rot2)
                w = w + _lane_perm(w, rot1)
                idx = (lane * 0 + (f2 * 2 * L + r)) + pair_off
                plsc.store_scatter(xxT, [idx], w, mask=pair_mask)
                return c
            lax.fori_loop(0, F // 2, pair_body, 0, unroll=4)

        def mlp1_body(j, c):
            accs = [zero_v] * 4
            for fc in range(FP // L):
                wv = w1_v[pl.ds(j * FP + fc * L, L)]
                for t in range(L):
                    accs[t & 3] = accs[t & 3] + _bcast_lane(wv, t, lane) * xxT[
                        pl.ds((fc * L + t) * L, L)]
            acc = (accs[0] + accs[1]) + (accs[2] + accs[3])
            hT[pl.ds(j * L, L)] = jnp.maximum(acc, 0.0)
            return c
        lax.fori_loop(0, RED, mlp1_body, 0)

        def mlp2_body(i, c):
            accs = [zero_v] * 4
            for jc in range(RP // L):
                wv = w2_v[pl.ds(i * RP + jc * L, L)]
                for t in range(L):
                    accs[t & 3] = accs[t & 3] + _bcast_lane(wv, t, lane) * hT[
                        pl.ds((jc * L + t) * L, L)]
            acc = (accs[0] + accs[1]) + (accs[2] + accs[3])
            sT[pl.ds(i * L, L)] = 1.0 / (1.0 + jnp.exp(-acc))
            return c
        lax.fori_loop(0, F, mlp2_body, 0)

        for r in range(G):
            def scale_body(f, c, r=r):
                sc = _bcast_lane(sT[pl.ds(f * L, L)], r, lane)
                base = f * SEG
                for t in range(4):
                    o = base + t * L
                    xbuf[r, pl.ds(o, L)] = xbuf[r, pl.ds(o, L)] * sc
                return c
            lax.fori_loop(0, F, scale_body, 0, unroll=2)

        pltpu.sync_copy(xbuf, out_hbm.at[pl.ds(row0, G)])
        return carry

    lax.fori_loop(0, NGROUPS, group_body, 0)


@functools.cache
def _sc_call():
    return pl.kernel(
        _sc_body,
        out_type=jax.ShapeDtypeStruct((B_SC, D), jnp.float32),
        mesh=plsc.VectorSubcoreMesh(core_axis_name="c", subcore_axis_name="s",
                                    num_cores=NC, num_subcores=NS),
        compiler_params=pltpu.CompilerParams(needs_layout_passes=False),
        scratch_types=[
            pltpu.VMEM((G, D), jnp.float32),
            pltpu.VMEM((RED * FP,), jnp.float32),
            pltpu.VMEM((F * RP,), jnp.float32),
            pltpu.VMEM((FP * L,), jnp.float32),
            pltpu.VMEM((RP * L,), jnp.float32),
            pltpu.VMEM((F * L,), jnp.float32),
        ],
    )


import numpy as np

_SM = np.repeat(np.eye(F, dtype=np.float32), SEG, axis=0) * (1.0 / SEG)
_RM = np.repeat(np.eye(F, dtype=np.float32), SEG, axis=1)


def _tc_body(x_ref, w1t_ref, w2t_ref, sm_ref, rm_ref, o_ref):
    xb = x_ref[...]
    x_hi = xb.astype(jnp.bfloat16)
    x_lo = (xb - x_hi.astype(jnp.float32)).astype(jnp.bfloat16)
    sm = sm_ref[...]
    xx = (jnp.dot(x_hi, sm, preferred_element_type=jnp.float32)
          + jnp.dot(x_lo, sm, preferred_element_type=jnp.float32))
    h = jnp.maximum(jnp.dot(xx, w1t_ref[...],
                            preferred_element_type=jnp.float32), 0.0)
    s = jax.nn.sigmoid(jnp.dot(h, w2t_ref[...],
                               preferred_element_type=jnp.float32))
    s_hi = s.astype(jnp.bfloat16)
    s_lo = (s - s_hi.astype(jnp.float32)).astype(jnp.bfloat16)
    rm = rm_ref[...]
    s_rep = (jnp.dot(s_hi, rm, preferred_element_type=jnp.float32)
             + jnp.dot(s_lo, rm, preferred_element_type=jnp.float32))
    o_ref[...] = xb * s_rep


def _tc_call(x, W1, W2):
    w1t = W1.T
    w2t = W2.T
    sm = jnp.asarray(_SM, dtype=jnp.bfloat16)
    rm = jnp.asarray(_RM, dtype=jnp.bfloat16)
    off = B_SC // TILE_B
    return pl.pallas_call(
        _tc_body,
        grid=(B_TC // TILE_B,),
        in_specs=[
            pl.BlockSpec((TILE_B, D), lambda i: (i + off, 0)),
            pl.BlockSpec((F, RED), lambda i: (0, 0)),
            pl.BlockSpec((RED, F), lambda i: (0, 0)),
            pl.BlockSpec((D, F), lambda i: (0, 0)),
            pl.BlockSpec((F, D), lambda i: (0, 0)),
        ],
        out_specs=pl.BlockSpec((TILE_B, D), lambda i: (i + off, 0)),
        out_shape=jax.ShapeDtypeStruct((B, D), jnp.float32),
    )(x, w1t, w2t, sm, rm)


def kernel(x, W1, W2):
    w1p = jnp.zeros((RED, FP), jnp.float32).at[:, :F].set(W1 * (1.0 / SEG))
    w2p = jnp.zeros((F, RP), jnp.float32).at[:, :RED].set(W2)
    out_tc = _tc_call(x, W1, W2)
    out_sc = _sc_call()(x, w1p.reshape(-1), w2p.reshape(-1))
    return lax.dynamic_update_slice(out_tc, out_sc, (0, 0))
